# baseline (speedup 1.0000x reference)
import functools

import jax
import jax.numpy as jnp
from jax import lax
from jax.experimental import pallas as pl
from jax.experimental.pallas import tpu as pltpu
from jax.experimental.pallas import tpu_sc as plsc

_NUM_USER = 1000000
_DIM = 32
_BATCH = 16384

_INFO = plsc.get_sparse_core_info()
_NC = _INFO.num_cores
_NS = _INFO.num_subcores
_NW = _NC * _NS
_B_PER_W = _BATCH // _NW


@functools.partial(
    pl.kernel,
    mesh=plsc.VectorSubcoreMesh(core_axis_name="c", subcore_axis_name="s"),
    out_type=jax.ShapeDtypeStruct((_BATCH, _DIM), jnp.float32),
    scratch_types=[
        pltpu.VMEM((_B_PER_W,), jnp.int32),
        pltpu.VMEM((_B_PER_W, _DIM), jnp.float32),
        pltpu.SemaphoreType.DMA,
    ],
)
def _sc_gather(table_hbm, idx_hbm, out_hbm, idx_v, rows_v, sem):
    wid = lax.axis_index("s") * _NC + lax.axis_index("c")
    base = wid * _B_PER_W
    pltpu.sync_copy(idx_hbm.at[pl.ds(base, _B_PER_W)], idx_v)

    def fire(g, carry):
        v = idx_v[pl.ds(g * 16, 16)]
        for j in range(16):
            row = v[j]
            pltpu.async_copy(
                table_hbm.at[pl.ds(row, 1), :],
                rows_v.at[pl.ds(g * 16 + j, 1), :],
                sem,
            )
        return carry

    lax.fori_loop(0, _B_PER_W // 16, fire, 0)

    def drain(i, carry):
        pltpu.make_async_copy(
            table_hbm.at[pl.ds(0, 1), :], rows_v.at[pl.ds(0, 1), :], sem
        ).wait()
        return carry

    lax.fori_loop(0, _B_PER_W, drain, 0)
    pltpu.sync_copy(rows_v, out_hbm.at[pl.ds(base, _B_PER_W)])


def kernel(user_fea, embedding_userId):
    idx = user_fea[:, 0].astype(jnp.int32)
    return _sc_gather(embedding_userId, idx)
